# baseline (speedup 1.0000x reference)
import functools

import jax
import jax.numpy as jnp
from jax.experimental import pallas as pl
from jax.experimental.pallas import tpu as pltpu


def _tail_body(y_ref, wl2_ref, bl2_ref, wlf_ref, blf_ref, y4_ref, xr_ref):
    y = y_ref[...]
    y4 = jnp.maximum(y @ wl2_ref[...] + bl2_ref[...], 0.0)
    y4_ref[...] = y4
    xr_ref[...] = y4 @ wlf_ref[...] + blf_ref[...]


def _tail(y, W_l2, b_l2, W_lf, b_lf):
    n, h = y.shape
    dout = W_lf.shape[1]
    blk = 2000
    grid = (n // blk,)
    return pl.pallas_call(
        _tail_body,
        grid=grid,
        in_specs=[
            pl.BlockSpec((blk, h), lambda i: (i, 0)),
            pl.BlockSpec((h, h), lambda i: (0, 0)),
            pl.BlockSpec((h,), lambda i: (0,)),
            pl.BlockSpec((h, dout), lambda i: (0, 0)),
            pl.BlockSpec((dout,), lambda i: (0,)),
        ],
        out_specs=[
            pl.BlockSpec((blk, h), lambda i: (i, 0)),
            pl.BlockSpec((blk, dout), lambda i: (i, 0)),
        ],
        out_shape=[
            jax.ShapeDtypeStruct((n, h), jnp.float32),
            jax.ShapeDtypeStruct((n, dout), jnp.float32),
        ],
    )(y, W_l2, b_l2, W_lf, b_lf)


def _gat_conv(x, src, dst, edge_attr_full, W, aS, aD, aE, We, b):
    n = x.shape[0]
    h = x @ W
    a_src = jnp.sum(h * aS, axis=-1)
    a_dst = jnp.sum(h * aD, axis=-1)
    a_e = jnp.sum((edge_attr_full @ We) * aE, axis=-1)
    alpha = jax.nn.leaky_relu(a_src[src] + a_dst[dst] + a_e, 0.2)
    amax = jax.ops.segment_max(alpha, dst, num_segments=n)
    ex = jnp.exp(alpha - amax[dst])
    denom = jax.ops.segment_sum(ex, dst, num_segments=n)
    coef = ex / (denom[dst] + 1e-16)
    out = jax.ops.segment_sum(coef[:, None] * h[src], dst, num_segments=n)
    return out + b


def kernel(x, edge_index, edge_attr, shift,
           W1, aS1, aD1, aE1, We1, b1,
           W2, aS2, aD2, aE2, We2, b2,
           W3, aS3, aD3, aE3, We3, b3,
           W_l2, b_l2, W_lf, b_lf):
    n = x.shape[0]
    loop = jnp.arange(n, dtype=edge_index.dtype)
    src = jnp.concatenate([edge_index[0], loop])
    dst = jnp.concatenate([edge_index[1], loop])
    mean_attr = jnp.mean(edge_attr, axis=0)
    ea = jnp.concatenate([edge_attr, jnp.tile(mean_attr[None, :], (n, 1))], axis=0)
    y0 = jax.nn.relu(_gat_conv(x, src, dst, ea, W1, aS1, aD1, aE1, We1, b1))
    y1 = jax.nn.relu(_gat_conv(y0, src, dst, ea, W2, aS2, aD2, aE2, We2, b2))
    y3 = jax.nn.relu(_gat_conv(y1 + y0, src, dst, ea, W3, aS3, aD3, aE3, We3, b3))
    y4, xr = _tail(y0 + y1 + y3, W_l2, b_l2, W_lf, b_lf)
    return (xr, y3)


# speedup vs baseline: 21.8256x; 21.8256x over previous
import functools

import jax
import jax.numpy as jnp
from jax import lax
from jax.experimental import pallas as pl
from jax.experimental.pallas import tpu as pltpu
from jax.experimental.pallas import tpu_sc as plsc

N = 10000
E = 320000
H = 128
NC = 2
NS = 16
NW = NC * NS
CH = E // NW
K = 80
NB = CH // K
SEG = 624
ZR = 104

@functools.lru_cache(maxsize=None)
def _mesh():
    return plsc.VectorSubcoreMesh(core_axis_name="c", subcore_axis_name="s")


def _sc_attn_body(h, asrc, adst, ae, src2, dst2, mvec, outp, denq,
                  rows, sidx, didx, asb, adb, aeb, exv, m_t,
                  zbuf, zd, acc, dspm, sem, sem2, sem3):
    c = lax.axis_index("c")
    s = lax.axis_index("s")
    wid = s * NC + c
    bbase = wid * NB

    def zrow(i, carry):
        for j in range(8):
            zbuf[i, pl.ds(j * 16, 16)] = jnp.zeros((16,), jnp.float32)
        return carry
    lax.fori_loop(0, ZR, zrow, 0)

    def zdz(i, carry):
        zd[pl.ds(i * 16, 16)] = jnp.zeros((16,), jnp.float32)
        return carry
    lax.fori_loop(0, 40, zdz, 0)

    for b in range(6):
        pltpu.sync_copy(zbuf, acc.at[pl.ds(s * SEG + b * ZR, ZR)])

    @pl.when(s == NS - 1)
    def _():
        pltpu.sync_copy(zbuf.at[pl.ds(0, 16)], acc.at[pl.ds(NS * SEG, 16)])

    @pl.when(s == 0)
    def _():
        for ch in range(15):
            pltpu.sync_copy(zd, dspm.at[pl.ds(ch * 640, 640)])
        pltpu.sync_copy(zd.at[pl.ds(0, 400)], dspm.at[pl.ds(9600, 400)])

    plsc.subcore_barrier()

    pltpu.sync_copy(mvec, m_t)
    m = m_t[...]

    def batch(b, carry):
        eb = bbase + b
        pltpu.sync_copy(src2.at[eb], sidx)
        pltpu.sync_copy(dst2.at[eb], didx)
        pltpu.sync_copy(ae.at[pl.ds(eb * K, K)], aeb)
        cph = pltpu.async_copy(h.at[sidx], rows, sem)
        cps = pltpu.async_copy(asrc.at[sidx], asb, sem2)
        cpd = pltpu.async_copy(adst.at[didx], adb, sem3)
        cps.wait()
        cpd.wait()

        def chunk(i, carry2):
            al = (asb[pl.ds(i * 16, 16)] + adb[pl.ds(i * 16, 16)]
                  + aeb[pl.ds(i * 16, 16)])
            al = jnp.where(al >= 0.0, al, al * jnp.float32(0.2))
            exv[pl.ds(i * 16, 16)] = jnp.exp(al - m)
            return carry2
        lax.fori_loop(0, K // 16, chunk, 0)

        cph.wait()

        def scale(i, carry2):
            ev = exv[pl.ds(i * 16, 16)]
            for k in range(16):
                e = ev[k]
                r = i * 16 + k
                for j in range(8):
                    rows[r, pl.ds(j * 16, 16)] = rows[r, pl.ds(j * 16, 16)] * e
            return carry2
        lax.fori_loop(0, K // 16, scale, 0)

        pltpu.sync_copy(rows, acc.at[didx], add=True)
        pltpu.sync_copy(exv, dspm.at[didx], add=True)
        return carry

    lax.fori_loop(0, NB, batch, 0)
    plsc.subcore_barrier()

    for b in range(6):
        r0 = s * SEG + b * ZR
        pltpu.sync_copy(acc.at[pl.ds(r0, ZR)], outp.at[c, pl.ds(r0, ZR)])

    @pl.when(s == NS - 1)
    def _():
        pltpu.sync_copy(acc.at[pl.ds(NS * SEG, 16)],
                        outp.at[c, pl.ds(NS * SEG, 16)])

    @pl.when(s == 0)
    def _():
        pltpu.sync_copy(dspm, denq.at[c])


def _sc_attn(h, asrc, adst, ae, src2, dst2, mvec):
    return pl.kernel(
        _sc_attn_body,
        mesh=_mesh(),
        out_type=(
            jax.ShapeDtypeStruct((NC, N, H), jnp.float32),
            jax.ShapeDtypeStruct((NC, N), jnp.float32),
        ),
        scratch_types=[
            pltpu.VMEM((K, H), jnp.float32),
            pltpu.VMEM((K,), jnp.int32),
            pltpu.VMEM((K,), jnp.int32),
            pltpu.VMEM((K,), jnp.float32),
            pltpu.VMEM((K,), jnp.float32),
            pltpu.VMEM((K,), jnp.float32),
            pltpu.VMEM((K,), jnp.float32),
            pltpu.VMEM((16,), jnp.float32),
            pltpu.VMEM((ZR, H), jnp.float32),
            pltpu.VMEM((640,), jnp.float32),
            pltpu.VMEM_SHARED((N, H), jnp.float32),
            pltpu.VMEM_SHARED((N,), jnp.float32),
            pltpu.SemaphoreType.DMA,
            pltpu.SemaphoreType.DMA,
            pltpu.SemaphoreType.DMA,
        ],
    )(h, asrc, adst, ae, src2, dst2, mvec)


_NBLK = 2000


def _dense_in_body(x_ref, w_ref, as_ref, ad_ref, h_ref, asd_ref, mx_ref):
    i = pl.program_id(0)
    h = x_ref[...] @ w_ref[...]
    h_ref[...] = h
    a_src = h @ as_ref[...]
    a_dst = h @ ad_ref[...]
    asd_ref[...] = jnp.stack([a_src, a_dst], axis=1)
    cur = jnp.stack([jnp.max(a_src), jnp.max(a_dst)]).reshape(1, 2)
    mx_ref[...] = jnp.where(i == 0, cur, jnp.maximum(mx_ref[...], cur))


def _dense_in(x, W, aS, aD):
    return pl.pallas_call(
        _dense_in_body,
        grid=(N // _NBLK,),
        in_specs=[
            pl.BlockSpec((_NBLK, H), lambda i: (i, 0)),
            pl.BlockSpec((H, H), lambda i: (0, 0)),
            pl.BlockSpec((H,), lambda i: (0,)),
            pl.BlockSpec((H,), lambda i: (0,)),
        ],
        out_specs=[
            pl.BlockSpec((_NBLK, H), lambda i: (i, 0)),
            pl.BlockSpec((_NBLK, 2), lambda i: (i, 0)),
            pl.BlockSpec((1, 2), lambda i: (0, 0)),
        ],
        out_shape=[
            jax.ShapeDtypeStruct((N, H), jnp.float32),
            jax.ShapeDtypeStruct((N, 2), jnp.float32),
            jax.ShapeDtypeStruct((1, 2), jnp.float32),
        ],
    )(x, W, aS, aD)


_EBLK = 16000


def _edge_pre_body(eat_ref, we1, ae1, we2, ae2, we3, ae3,
                   ae3_ref, c3_ref, sum_ref, max_ref):
    i = pl.program_id(0)
    c1 = we1[...] @ ae1[...]
    c2 = we2[...] @ ae2[...]
    c3 = we3[...] @ ae3[...]
    ea0 = eat_ref[0, :]
    ea1 = eat_ref[1, :]
    rows = jnp.stack([ea0 * c1[0] + ea1 * c1[1],
                      ea0 * c2[0] + ea1 * c2[1],
                      ea0 * c3[0] + ea1 * c3[1]])
    ae3_ref[...] = rows
    c3_ref[...] = jnp.stack([c1, c2, c3], axis=1)
    cur_s = jnp.stack([jnp.sum(ea0), jnp.sum(ea1)]).reshape(1, 2)
    cur_m = jnp.max(rows, axis=1).reshape(1, 3)
    sum_ref[...] = jnp.where(i == 0, cur_s, sum_ref[...] + cur_s)
    max_ref[...] = jnp.where(i == 0, cur_m, jnp.maximum(max_ref[...], cur_m))


def _edge_pre(eaT, We1, aE1, We2, aE2, We3, aE3):
    small = [
        pl.BlockSpec((2, H), lambda i: (0, 0)),
        pl.BlockSpec((H,), lambda i: (0,)),
    ] * 3
    return pl.pallas_call(
        _edge_pre_body,
        grid=(E // _EBLK,),
        in_specs=[pl.BlockSpec((2, _EBLK), lambda i: (0, i))] + small,
        out_specs=[
            pl.BlockSpec((3, _EBLK), lambda i: (0, i)),
            pl.BlockSpec((2, 3), lambda i: (0, 0)),
            pl.BlockSpec((1, 2), lambda i: (0, 0)),
            pl.BlockSpec((1, 3), lambda i: (0, 0)),
        ],
        out_shape=[
            jax.ShapeDtypeStruct((3, E), jnp.float32),
            jax.ShapeDtypeStruct((2, 3), jnp.float32),
            jax.ShapeDtypeStruct((1, 2), jnp.float32),
            jax.ShapeDtypeStruct((1, 3), jnp.float32),
        ],
    )(eaT, We1, aE1, We2, aE2, We3, aE3)


def _finish_body(has_extra, outp_ref, denq_ref, asd_ref, h_ref, b_ref,
                 sc_ref, *rest):
    if has_extra:
        r_ref, y_ref, xn_ref = rest
    else:
        (y_ref,) = rest
    mm = sc_ref[0, 0]
    ae_loop = sc_ref[0, 1]
    asd = asd_ref[...]
    a = asd[:, 0] + asd[:, 1] + ae_loop
    a = jnp.where(a >= 0.0, a, a * jnp.float32(0.2))
    wl = jnp.exp(a - mm)
    numer = outp_ref[0] + outp_ref[1] + wl[:, None] * h_ref[...]
    dq = denq_ref[...]
    den = dq[:, 0] + dq[:, 1] + wl
    y = jnp.maximum(numer / den[:, None] + b_ref[...], 0.0)
    y_ref[...] = y
    if has_extra:
        xn_ref[...] = y + r_ref[...]


def _finish(outp, denq, asd, h, b, scal, extra=None):
    has_extra = extra is not None
    in_specs = [
        pl.BlockSpec((2, _NBLK, H), lambda i: (0, i, 0)),
        pl.BlockSpec((_NBLK, 2), lambda i: (i, 0)),
        pl.BlockSpec((_NBLK, 2), lambda i: (i, 0)),
        pl.BlockSpec((_NBLK, H), lambda i: (i, 0)),
        pl.BlockSpec((H,), lambda i: (0,)),
        pl.BlockSpec((1, 8), lambda i: (0, 0)),
    ]
    out_specs = [pl.BlockSpec((_NBLK, H), lambda i: (i, 0))]
    out_shape = [jax.ShapeDtypeStruct((N, H), jnp.float32)]
    args = [outp, denq, asd, h, b, scal]
    if has_extra:
        in_specs.append(pl.BlockSpec((_NBLK, H), lambda i: (i, 0)))
        out_specs.append(pl.BlockSpec((_NBLK, H), lambda i: (i, 0)))
        out_shape.append(jax.ShapeDtypeStruct((N, H), jnp.float32))
        args.append(extra)
    res = pl.pallas_call(
        functools.partial(_finish_body, has_extra),
        grid=(N // _NBLK,),
        in_specs=in_specs,
        out_specs=out_specs,
        out_shape=out_shape,
    )(*args)
    return res if has_extra else (res[0], None)


def _tail_body(y_ref, wl2_ref, bl2_ref, wlf_ref, blf_ref, xr_ref):
    y4 = jnp.maximum(y_ref[...] @ wl2_ref[...] + bl2_ref[...], 0.0)
    xr_ref[...] = y4 @ wlf_ref[...] + blf_ref[...]


def _tail(t, W_l2, b_l2, W_lf, b_lf):
    dout = W_lf.shape[1]
    return pl.pallas_call(
        _tail_body,
        grid=(N // _NBLK,),
        in_specs=[
            pl.BlockSpec((_NBLK, H), lambda i: (i, 0)),
            pl.BlockSpec((H, H), lambda i: (0, 0)),
            pl.BlockSpec((H,), lambda i: (0,)),
            pl.BlockSpec((H, dout), lambda i: (0, 0)),
            pl.BlockSpec((dout,), lambda i: (0,)),
        ],
        out_specs=pl.BlockSpec((_NBLK, dout), lambda i: (i, 0)),
        out_shape=jax.ShapeDtypeStruct((N, dout), jnp.float32),
    )(t, W_l2, b_l2, W_lf, b_lf)


def _layer(x, src2, dst2, ae_l, aemax_l, ae_loop_l,
           W, aS, aD, b, extra):
    h, asd, mx = _dense_in(x, W, aS, aD)
    M = mx[0, 0] + mx[0, 1] + jnp.maximum(aemax_l, ae_loop_l)
    mvec = jnp.full((16,), M, jnp.float32)
    outp, denq = _sc_attn(h, asd[:, 0], asd[:, 1], ae_l, src2, dst2, mvec)
    scal = jnp.zeros((1, 8), jnp.float32).at[0, 0].set(M).at[0, 1].set(ae_loop_l)
    return _finish(outp, denq.T, asd, h, b, scal, extra)


def kernel(x, edge_index, edge_attr, shift,
           W1, aS1, aD1, aE1, We1, b1,
           W2, aS2, aD2, aE2, We2, b2,
           W3, aS3, aD3, aE3, We3, b3,
           W_l2, b_l2, W_lf, b_lf):
    src2 = edge_index[0].reshape(E // K, K)
    dst2 = edge_index[1].reshape(E // K, K)
    eaT = edge_attr.T

    ae3, c3, easum, aemax = _edge_pre(eaT, We1, aE1, We2, aE2, We3, aE3)
    mean_attr = easum[0] / jnp.float32(E)
    ae_loops = mean_attr @ c3

    y0, _ = _layer(x, src2, dst2, ae3[0], aemax[0, 0], ae_loops[0],
                   W1, aS1, aD1, b1, None)
    y1, x2 = _layer(y0, src2, dst2, ae3[1], aemax[0, 1], ae_loops[1],
                    W2, aS2, aD2, b2, y0)
    y3, t = _layer(x2, src2, dst2, ae3[2], aemax[0, 2], ae_loops[2],
                   W3, aS3, aD3, b3, x2)
    xr = _tail(t, W_l2, b_l2, W_lf, b_lf)
    return (xr, y3)


# speedup vs baseline: 31.2150x; 1.4302x over previous
import functools

import jax
import jax.numpy as jnp
from jax import lax
from jax.experimental import pallas as pl
from jax.experimental.pallas import tpu as pltpu
from jax.experimental.pallas import tpu_sc as plsc

N = 10000
E = 320000
H = 128
NC = 2
NS = 16
NW = NC * NS
CH = E // NW
K = 200
NB = CH // K
SEG = 624
ZR = 104

@functools.lru_cache(maxsize=None)
def _mesh():
    return plsc.VectorSubcoreMesh(core_axis_name="c", subcore_axis_name="s")


def _sc_attn_body(h, asrc, adst, ae, src2, dst2, mvec, outp, denq,
                  rows, sidx, didx, asb, adb, aeb, exv, m_t,
                  zbuf, zd, acc, dspm, sem, sem2, sem3):
    c = lax.axis_index("c")
    s = lax.axis_index("s")
    wid = s * NC + c
    bbase = wid * NB

    def zrow(i, carry):
        for j in range(8):
            zbuf[i, pl.ds(j * 16, 16)] = jnp.zeros((16,), jnp.float32)
        return carry
    lax.fori_loop(0, ZR, zrow, 0)

    def zdz(i, carry):
        zd[pl.ds(i * 16, 16)] = jnp.zeros((16,), jnp.float32)
        return carry
    lax.fori_loop(0, 40, zdz, 0)

    for b in range(6):
        pltpu.sync_copy(zbuf, acc.at[pl.ds(s * SEG + b * ZR, ZR)])

    @pl.when(s == NS - 1)
    def _():
        pltpu.sync_copy(zbuf.at[pl.ds(0, 16)], acc.at[pl.ds(NS * SEG, 16)])

    @pl.when(s == 0)
    def _():
        for ch in range(15):
            pltpu.sync_copy(zd, dspm.at[pl.ds(ch * 640, 640)])
        pltpu.sync_copy(zd.at[pl.ds(0, 400)], dspm.at[pl.ds(9600, 400)])

    plsc.subcore_barrier()

    pltpu.sync_copy(mvec, m_t)
    m = m_t[...]

    def batch(b, carry):
        eb = bbase + b
        pltpu.sync_copy(src2.at[eb], sidx)
        pltpu.sync_copy(dst2.at[eb], didx)
        pltpu.sync_copy(ae.at[pl.ds(eb * K, K)], aeb)
        cph = pltpu.async_copy(h.at[sidx], rows, sem)
        cps = pltpu.async_copy(asrc.at[sidx], asb, sem2)
        cpd = pltpu.async_copy(adst.at[didx], adb, sem3)
        cps.wait()
        cpd.wait()

        def chunk(i, carry2):
            al = (asb[pl.ds(i * 16, 16)] + adb[pl.ds(i * 16, 16)]
                  + aeb[pl.ds(i * 16, 16)])
            al = jnp.where(al >= 0.0, al, al * jnp.float32(0.2))
            exv[pl.ds(i * 16, 16)] = jnp.exp(al - m)
            return carry2
        lax.fori_loop(0, K // 16, chunk, 0)

        cph.wait()

        def scale(i, carry2):
            ev = exv[pl.ds(i * 16, 16)]
            for k in range(16):
                e = ev[k]
                r = i * 16 + k
                for j in range(8):
                    rows[r, pl.ds(j * 16, 16)] = rows[r, pl.ds(j * 16, 16)] * e
            return carry2
        lax.fori_loop(0, K // 16, scale, 0)

        pltpu.sync_copy(rows, acc.at[didx], add=True)
        pltpu.sync_copy(exv, dspm.at[didx], add=True)
        return carry

    lax.fori_loop(0, NB, batch, 0)
    plsc.subcore_barrier()

    for b in range(6):
        r0 = s * SEG + b * ZR
        pltpu.sync_copy(acc.at[pl.ds(r0, ZR)], outp.at[c, pl.ds(r0, ZR)])

    @pl.when(s == NS - 1)
    def _():
        pltpu.sync_copy(acc.at[pl.ds(NS * SEG, 16)],
                        outp.at[c, pl.ds(NS * SEG, 16)])

    @pl.when(s == 0)
    def _():
        pltpu.sync_copy(dspm, denq.at[c])


def _sc_attn(h, asrc, adst, ae, src2, dst2, mvec):
    return pl.kernel(
        _sc_attn_body,
        mesh=_mesh(),
        out_type=(
            jax.ShapeDtypeStruct((NC, N, H), jnp.float32),
            jax.ShapeDtypeStruct((NC, N), jnp.float32),
        ),
        scratch_types=[
            pltpu.VMEM((K, H), jnp.float32),
            pltpu.VMEM((K,), jnp.int32),
            pltpu.VMEM((K,), jnp.int32),
            pltpu.VMEM((K,), jnp.float32),
            pltpu.VMEM((K,), jnp.float32),
            pltpu.VMEM((K,), jnp.float32),
            pltpu.VMEM((K,), jnp.float32),
            pltpu.VMEM((16,), jnp.float32),
            pltpu.VMEM((ZR, H), jnp.float32),
            pltpu.VMEM((640,), jnp.float32),
            pltpu.VMEM_SHARED((N, H), jnp.float32),
            pltpu.VMEM_SHARED((N,), jnp.float32),
            pltpu.SemaphoreType.DMA,
            pltpu.SemaphoreType.DMA,
            pltpu.SemaphoreType.DMA,
        ],
    )(h, asrc, adst, ae, src2, dst2, mvec)


_NBLK = 2000


def _dense_in_body(x_ref, w_ref, as_ref, ad_ref, h_ref, asd_ref, mx_ref):
    i = pl.program_id(0)
    h = x_ref[...] @ w_ref[...]
    h_ref[...] = h
    a_src = h @ as_ref[...]
    a_dst = h @ ad_ref[...]
    asd_ref[...] = jnp.stack([a_src, a_dst], axis=1)
    cur = jnp.stack([jnp.max(a_src), jnp.max(a_dst)]).reshape(1, 2)
    mx_ref[...] = jnp.where(i == 0, cur, jnp.maximum(mx_ref[...], cur))


def _dense_in(x, W, aS, aD):
    return pl.pallas_call(
        _dense_in_body,
        grid=(N // _NBLK,),
        in_specs=[
            pl.BlockSpec((_NBLK, H), lambda i: (i, 0)),
            pl.BlockSpec((H, H), lambda i: (0, 0)),
            pl.BlockSpec((H,), lambda i: (0,)),
            pl.BlockSpec((H,), lambda i: (0,)),
        ],
        out_specs=[
            pl.BlockSpec((_NBLK, H), lambda i: (i, 0)),
            pl.BlockSpec((_NBLK, 2), lambda i: (i, 0)),
            pl.BlockSpec((1, 2), lambda i: (0, 0)),
        ],
        out_shape=[
            jax.ShapeDtypeStruct((N, H), jnp.float32),
            jax.ShapeDtypeStruct((N, 2), jnp.float32),
            jax.ShapeDtypeStruct((1, 2), jnp.float32),
        ],
    )(x, W, aS, aD)


_EBLK = 16000


def _edge_pre_body(eat_ref, we1, ae1, we2, ae2, we3, ae3,
                   ae3_ref, c3_ref, sum_ref, max_ref):
    i = pl.program_id(0)
    c1 = we1[...] @ ae1[...]
    c2 = we2[...] @ ae2[...]
    c3 = we3[...] @ ae3[...]
    ea0 = eat_ref[0, :]
    ea1 = eat_ref[1, :]
    rows = jnp.stack([ea0 * c1[0] + ea1 * c1[1],
                      ea0 * c2[0] + ea1 * c2[1],
                      ea0 * c3[0] + ea1 * c3[1]])
    ae3_ref[...] = rows
    c3_ref[...] = jnp.stack([c1, c2, c3], axis=1)
    cur_s = jnp.stack([jnp.sum(ea0), jnp.sum(ea1)]).reshape(1, 2)
    cur_m = jnp.max(rows, axis=1).reshape(1, 3)
    sum_ref[...] = jnp.where(i == 0, cur_s, sum_ref[...] + cur_s)
    max_ref[...] = jnp.where(i == 0, cur_m, jnp.maximum(max_ref[...], cur_m))


def _edge_pre(eaT, We1, aE1, We2, aE2, We3, aE3):
    small = [
        pl.BlockSpec((2, H), lambda i: (0, 0)),
        pl.BlockSpec((H,), lambda i: (0,)),
    ] * 3
    return pl.pallas_call(
        _edge_pre_body,
        grid=(E // _EBLK,),
        in_specs=[pl.BlockSpec((2, _EBLK), lambda i: (0, i))] + small,
        out_specs=[
            pl.BlockSpec((3, _EBLK), lambda i: (0, i)),
            pl.BlockSpec((2, 3), lambda i: (0, 0)),
            pl.BlockSpec((1, 2), lambda i: (0, 0)),
            pl.BlockSpec((1, 3), lambda i: (0, 0)),
        ],
        out_shape=[
            jax.ShapeDtypeStruct((3, E), jnp.float32),
            jax.ShapeDtypeStruct((2, 3), jnp.float32),
            jax.ShapeDtypeStruct((1, 2), jnp.float32),
            jax.ShapeDtypeStruct((1, 3), jnp.float32),
        ],
    )(eaT, We1, aE1, We2, aE2, We3, aE3)


def _finish_body(has_extra, outp_ref, denq_ref, asd_ref, h_ref, b_ref,
                 sc_ref, *rest):
    if has_extra:
        r_ref, y_ref, xn_ref = rest
    else:
        (y_ref,) = rest
    mm = sc_ref[0, 0]
    ae_loop = sc_ref[0, 1]
    asd = asd_ref[...]
    a = asd[:, 0] + asd[:, 1] + ae_loop
    a = jnp.where(a >= 0.0, a, a * jnp.float32(0.2))
    wl = jnp.exp(a - mm)
    numer = outp_ref[0] + outp_ref[1] + wl[:, None] * h_ref[...]
    dq = denq_ref[...]
    den = dq[:, 0] + dq[:, 1] + wl
    y = jnp.maximum(numer / den[:, None] + b_ref[...], 0.0)
    y_ref[...] = y
    if has_extra:
        xn_ref[...] = y + r_ref[...]


def _finish(outp, denq, asd, h, b, scal, extra=None):
    has_extra = extra is not None
    in_specs = [
        pl.BlockSpec((2, _NBLK, H), lambda i: (0, i, 0)),
        pl.BlockSpec((_NBLK, 2), lambda i: (i, 0)),
        pl.BlockSpec((_NBLK, 2), lambda i: (i, 0)),
        pl.BlockSpec((_NBLK, H), lambda i: (i, 0)),
        pl.BlockSpec((H,), lambda i: (0,)),
        pl.BlockSpec((1, 8), lambda i: (0, 0)),
    ]
    out_specs = [pl.BlockSpec((_NBLK, H), lambda i: (i, 0))]
    out_shape = [jax.ShapeDtypeStruct((N, H), jnp.float32)]
    args = [outp, denq, asd, h, b, scal]
    if has_extra:
        in_specs.append(pl.BlockSpec((_NBLK, H), lambda i: (i, 0)))
        out_specs.append(pl.BlockSpec((_NBLK, H), lambda i: (i, 0)))
        out_shape.append(jax.ShapeDtypeStruct((N, H), jnp.float32))
        args.append(extra)
    res = pl.pallas_call(
        functools.partial(_finish_body, has_extra),
        grid=(N // _NBLK,),
        in_specs=in_specs,
        out_specs=out_specs,
        out_shape=out_shape,
    )(*args)
    return res if has_extra else (res[0], None)


def _tail_body(y_ref, wl2_ref, bl2_ref, wlf_ref, blf_ref, xr_ref):
    y4 = jnp.maximum(y_ref[...] @ wl2_ref[...] + bl2_ref[...], 0.0)
    xr_ref[...] = y4 @ wlf_ref[...] + blf_ref[...]


def _tail(t, W_l2, b_l2, W_lf, b_lf):
    dout = W_lf.shape[1]
    return pl.pallas_call(
        _tail_body,
        grid=(N // _NBLK,),
        in_specs=[
            pl.BlockSpec((_NBLK, H), lambda i: (i, 0)),
            pl.BlockSpec((H, H), lambda i: (0, 0)),
            pl.BlockSpec((H,), lambda i: (0,)),
            pl.BlockSpec((H, dout), lambda i: (0, 0)),
            pl.BlockSpec((dout,), lambda i: (0,)),
        ],
        out_specs=pl.BlockSpec((_NBLK, dout), lambda i: (i, 0)),
        out_shape=jax.ShapeDtypeStruct((N, dout), jnp.float32),
    )(t, W_l2, b_l2, W_lf, b_lf)


def _layer(x, src2, dst2, ae_l, aemax_l, ae_loop_l,
           W, aS, aD, b, extra):
    h, asd, mx = _dense_in(x, W, aS, aD)
    M = mx[0, 0] + mx[0, 1] + jnp.maximum(aemax_l, ae_loop_l)
    mvec = jnp.full((16,), M, jnp.float32)
    outp, denq = _sc_attn(h, asd[:, 0], asd[:, 1], ae_l, src2, dst2, mvec)
    scal = jnp.zeros((1, 8), jnp.float32).at[0, 0].set(M).at[0, 1].set(ae_loop_l)
    return _finish(outp, denq.T, asd, h, b, scal, extra)


def kernel(x, edge_index, edge_attr, shift,
           W1, aS1, aD1, aE1, We1, b1,
           W2, aS2, aD2, aE2, We2, b2,
           W3, aS3, aD3, aE3, We3, b3,
           W_l2, b_l2, W_lf, b_lf):
    src2 = edge_index[0].reshape(E // K, K)
    dst2 = edge_index[1].reshape(E // K, K)
    eaT = edge_attr.T

    ae3, c3, easum, aemax = _edge_pre(eaT, We1, aE1, We2, aE2, We3, aE3)
    mean_attr = easum[0] / jnp.float32(E)
    ae_loops = mean_attr @ c3

    y0, _ = _layer(x, src2, dst2, ae3[0], aemax[0, 0], ae_loops[0],
                   W1, aS1, aD1, b1, None)
    y1, x2 = _layer(y0, src2, dst2, ae3[1], aemax[0, 1], ae_loops[1],
                    W2, aS2, aD2, b2, y0)
    y3, t = _layer(x2, src2, dst2, ae3[2], aemax[0, 2], ae_loops[2],
                   W3, aS3, aD3, b3, x2)
    xr = _tail(t, W_l2, b_l2, W_lf, b_lf)
    return (xr, y3)
